# Initial kernel scaffold; baseline (speedup 1.0000x reference)
import jax
import jax.numpy as jnp
from jax.experimental import pallas as pl


def kernel(x, edge_index, adj_values, Wg, bg, Wm1, bm1, Wm2, bm2, Ws1, bs1, Ws2, bs2, Wnb, bnb, Wsf, bsf, Watt, batt, Wa1, ba1, Wa2, Wh, bh):
    raise NotImplementedError("write your pallas kernel here")


# speedup vs baseline: 4.9595x; 4.9595x over previous
import functools

import jax
import jax.numpy as jnp
from jax import lax
from jax.experimental import pallas as pl
from jax.experimental.pallas import tpu as pltpu
from jax.experimental.pallas import tpu_sc as plsc

N = 10000
D = 128
H = 256
Z = 64
E = 320000
C = 10
GAMMA = -0.5
ZETA = 1.1
TEMP = 0.5
LOGC = -0.7884573603642702

NC = 2
NS = 16
NW = NC * NS
ET = E // NW
EJ = 79
ETP = EJ * 128
AP = 10240
AT = AP // NS
RSP = 10240
RST = RSP // NS

def _mesh():
    return plsc.VectorSubcoreMesh(
        core_axis_name="c", subcore_axis_name="s",
        num_cores=NC, num_subcores=NS)


def _t1_body(x_ref, wg_ref, bg_ref, wnb_ref, bnb_ref, wsf_ref, bsf_ref,
             watt_ref, hm4_ref, uv_ref):
    xb = x_ref[...]
    dn = (((1,), (1,)), ((), ()))
    hm = lax.dot_general(xb, wg_ref[...], dn,
                         preferred_element_type=jnp.float32) + bg_ref[...]
    for q in range(4):
        hq = hm[:, q * 64:(q + 1) * 64]
        hm4_ref[q] = jnp.concatenate([hq, hq], axis=1)
    a1 = jax.nn.relu(lax.dot_general(xb, wnb_ref[...], dn,
                                     preferred_element_type=jnp.float32)
                     + bnb_ref[...])
    a2 = jax.nn.relu(lax.dot_general(xb, wsf_ref[...], dn,
                                     preferred_element_type=jnp.float32)
                     + bsf_ref[...])
    wa = watt_ref[...].astype(jnp.bfloat16)
    u = lax.dot_general(a1.astype(jnp.bfloat16), wa[0, :H],
                        (((1,), (0,)), ((), ())),
                        preferred_element_type=jnp.float32)
    v = lax.dot_general(a2.astype(jnp.bfloat16), wa[0, H:],
                        (((1,), (0,)), ((), ())),
                        preferred_element_type=jnp.float32)
    uv_ref[...] = jnp.stack([u, v], axis=1)


def _t1(x, Wg, bg, Wnb, bnb, Wsf, bsf, Watt):
    bn = 2000
    grid = (N // bn,)
    full = lambda a: pl.BlockSpec(a.shape, lambda i: (0,) * a.ndim)
    return pl.pallas_call(
        _t1_body,
        grid=grid,
        in_specs=[pl.BlockSpec((bn, D), lambda i: (i, 0)),
                  full(Wg), full(bg), full(Wnb), full(bnb), full(Wsf),
                  full(bsf), full(Watt)],
        out_specs=[pl.BlockSpec((4, bn, 128), lambda i: (0, i, 0)),
                   pl.BlockSpec((bn, 2), lambda i: (i, 0))],
        out_shape=[jax.ShapeDtypeStruct((4, N, 128), jnp.float32),
                   jax.ShapeDtypeStruct((N, 2), jnp.float32)],
    )(x, Wg, bg, Wnb, bnb, Wsf, bsf, Watt)


def _sca_body(r_hbm, c_hbm, a_hbm, u_hbm, v_hbm, zn_hbm, b_hbm,
              mv_hbm, rs_hbm, l0_hbm,
              ridx, cidx, adjv, ugv, vgv, mvv, l0v, bv, rsum_sh, sem):
    c = lax.axis_index("c")
    s = lax.axis_index("s")
    wid = c * NS + s
    pltpu.sync_copy(zn_hbm, rsum_sh.at[pl.ds(s * RST, RST)])
    pltpu.sync_copy(b_hbm, bv)
    pltpu.sync_copy(r_hbm.at[wid], ridx)
    pltpu.sync_copy(c_hbm.at[wid], cidx)
    pltpu.sync_copy(a_hbm.at[wid], adjv)
    def gfire(j, carry):
        pltpu.async_copy(u_hbm.at[ridx.at[j]], ugv.at[j], sem)
        pltpu.async_copy(v_hbm.at[cidx.at[j]], vgv.at[j], sem)
        return carry

    lax.fori_loop(0, EJ, gfire, 0)

    def gdrain(j, carry):
        pltpu.make_async_copy(u_hbm.at[ridx.at[j]], ugv.at[j], sem).wait()
        pltpu.make_async_copy(v_hbm.at[cidx.at[j]], vgv.at[j], sem).wait()
        return carry

    lax.fori_loop(0, EJ, gdrain, 0)

    b16 = bv[...]

    def step(i, l0acc):
        j = i // 8
        k = (i % 8) * 16
        w = (ugv[j, pl.ds(k, 16)] + vgv[j, pl.ds(k, 16)]) + b16
        gate = 1.0 / (1.0 + jnp.exp(-w))
        m = jnp.clip(gate * (ZETA - GAMMA) + GAMMA, 0.0, 1.0)
        mvv[j, pl.ds(k, 16)] = adjv[j, pl.ds(k, 16)] * m
        return l0acc + 1.0 / (1.0 + jnp.exp(-(w - LOGC)))

    l0acc = lax.fori_loop(0, ET // 16, step, jnp.zeros((16,), jnp.float32))

    def step_pad(i, carry):
        j = i // 8
        k = (i % 8) * 16
        w = (ugv[j, pl.ds(k, 16)] + vgv[j, pl.ds(k, 16)]) + b16
        gate = 1.0 / (1.0 + jnp.exp(-w))
        m = jnp.clip(gate * (ZETA - GAMMA) + GAMMA, 0.0, 1.0)
        mvv[j, pl.ds(k, 16)] = adjv[j, pl.ds(k, 16)] * m
        return carry

    lax.fori_loop(ET // 16, ETP // 16, step_pad, 0)

    pltpu.sync_copy(mvv, mv_hbm.at[wid])
    plsc.subcore_barrier()

    def scat(j, carry):
        pltpu.async_copy(mvv.at[j], rsum_sh.at[ridx.at[j]], sem, add=True)
        return carry

    lax.fori_loop(0, EJ, scat, 0)

    def drain(j, carry):
        pltpu.make_async_copy(mvv.at[j], rsum_sh.at[ridx.at[j]], sem).wait()
        return carry

    lax.fori_loop(0, EJ, drain, 0)
    plsc.subcore_barrier()
    pltpu.sync_copy(rsum_sh.at[pl.ds(s * RST, RST)],
                    rs_hbm.at[c, pl.ds(s * RST, RST)])
    l0v[...] = l0acc
    pltpu.sync_copy(l0v, l0_hbm.at[wid])


def _sca_kw():
    return dict(
        out_type=(jax.ShapeDtypeStruct((NW, EJ, 128), jnp.float32),
                  jax.ShapeDtypeStruct((NC, RSP), jnp.float32),
                  jax.ShapeDtypeStruct((NW, 16), jnp.float32)),
        mesh=_mesh(),
        scratch_types=[pltpu.VMEM((EJ, 128), jnp.int32),
                       pltpu.VMEM((EJ, 128), jnp.int32),
                       pltpu.VMEM((EJ, 128), jnp.float32),
                       pltpu.VMEM((EJ, 128), jnp.float32),
                       pltpu.VMEM((EJ, 128), jnp.float32),
                       pltpu.VMEM((EJ, 128), jnp.float32),
                       pltpu.VMEM((16,), jnp.float32),
                       pltpu.VMEM((16,), jnp.float32),
                       pltpu.MemorySpace.VMEM_SHARED((RSP,), jnp.float32),
                       pltpu.SemaphoreType.DMA])


@functools.cache
def _sca():
    return pl.kernel(_sca_body, **_sca_kw())


def _tmid_body(rs_ref, l0_ref, dinv_ref, l0o_ref):
    t = rs_ref[0] + rs_ref[1] + 1e-10
    dinv_ref[...] = lax.rsqrt(t)
    l0o_ref[...] = jnp.sum(l0_ref[...]).reshape(1, 1) * (1.0 / E)


def _tmid(rs, l0p):
    return pl.pallas_call(
        _tmid_body,
        out_shape=[jax.ShapeDtypeStruct((80, 128), jnp.float32),
                   jax.ShapeDtypeStruct((1, 1), jnp.float32)],
    )(rs.reshape(NC, 80, 128), l0p)


def _scn_body(r_hbm, c_hbm, mv_hbm, dinv_hbm, nv_hbm,
              ridx, cidx, mvv, drv, dcv, sem):
    c = lax.axis_index("c")
    s = lax.axis_index("s")
    wid = c * NS + s
    pltpu.sync_copy(r_hbm.at[wid], ridx)
    pltpu.sync_copy(c_hbm.at[wid], cidx)
    pltpu.sync_copy(mv_hbm.at[wid], mvv)

    def gfire(j, carry):
        pltpu.async_copy(dinv_hbm.at[ridx.at[j]], drv.at[j], sem)
        pltpu.async_copy(dinv_hbm.at[cidx.at[j]], dcv.at[j], sem)
        return carry

    lax.fori_loop(0, EJ, gfire, 0)

    def gdrain(j, carry):
        pltpu.make_async_copy(dinv_hbm.at[ridx.at[j]], drv.at[j], sem).wait()
        pltpu.make_async_copy(dinv_hbm.at[cidx.at[j]], dcv.at[j], sem).wait()
        return carry

    lax.fori_loop(0, EJ, gdrain, 0)

    def nstep(i, carry):
        j = i // 8
        k = (i % 8) * 16
        mvv[j, pl.ds(k, 16)] = (mvv[j, pl.ds(k, 16)]
                                * drv[j, pl.ds(k, 16)] * dcv[j, pl.ds(k, 16)])
        return carry

    lax.fori_loop(0, ETP // 16, nstep, 0)
    pltpu.sync_copy(mvv, nv_hbm.at[wid])


def _scn_kw():
    return dict(
        out_type=jax.ShapeDtypeStruct((NW, EJ, 128), jnp.float32),
        mesh=_mesh(),
        scratch_types=[pltpu.VMEM((EJ, 128), jnp.int32),
                       pltpu.VMEM((EJ, 128), jnp.int32),
                       pltpu.VMEM((EJ, 128), jnp.float32),
                       pltpu.VMEM((EJ, 128), jnp.float32),
                       pltpu.VMEM((EJ, 128), jnp.float32),
                       pltpu.SemaphoreType.DMA])


@functools.cache
def _scn():
    return pl.kernel(_scn_body, **_scn_kw())


def _scb_body(r_hbm, c_hbm, a_hbm, nv_hbm,
              hq0, hq1, hq2, hq3, za_hbm,
              s_hbm,
              ridx, cidx, av, nvb, gbuf, acc, sem):
    c = lax.axis_index("c")
    s = lax.axis_index("s")
    wid = c * NS + s
    pltpu.sync_copy(r_hbm.at[wid], ridx)
    pltpu.sync_copy(c_hbm.at[wid], cidx)

    for q, hq in enumerate((hq0, hq1, hq2, hq3)):
        pltpu.sync_copy(za_hbm, acc.at[pl.ds(s * AT, AT)])
        plsc.subcore_barrier()

        def chunk(j, carry):
            pltpu.sync_copy(a_hbm.at[wid, j], av)
            pltpu.sync_copy(nv_hbm.at[wid, j], nvb)
            pltpu.async_copy(hq.at[cidx.at[j]], gbuf, sem).wait()

            def estep(kb, carry2):
                a16 = av[pl.ds(kb * 16, 16)]
                n16 = nvb[pl.ds(kb * 16, 16)]
                for lane in range(16):
                    e = kb * 16 + lane
                    a = a16[lane]
                    nv = n16[lane]
                    for kk in range(4):
                        gbuf[e, pl.ds(kk * 16, 16)] = (
                            gbuf[e, pl.ds(kk * 16, 16)] * a)
                    for kk in range(4, 8):
                        gbuf[e, pl.ds(kk * 16, 16)] = (
                            gbuf[e, pl.ds(kk * 16, 16)] * nv)
                return carry2

            lax.fori_loop(0, 8, estep, 0)
            pltpu.sync_copy(gbuf, acc.at[ridx.at[j]], add=True)
            return carry

        lax.fori_loop(0, EJ, chunk, 0)
        plsc.subcore_barrier()
        pltpu.sync_copy(acc.at[pl.ds(s * AT, AT)],
                        s_hbm.at[c, q, pl.ds(s * AT, AT)])


def _scb_kw():
    return dict(
        out_type=jax.ShapeDtypeStruct((NC, 4, AP, 128), jnp.float32),
        mesh=_mesh(),
        scratch_types=[pltpu.VMEM((EJ, 128), jnp.int32),
                       pltpu.VMEM((EJ, 128), jnp.int32),
                       pltpu.VMEM((128,), jnp.float32),
                       pltpu.VMEM((128,), jnp.float32),
                       pltpu.VMEM((128, 128), jnp.float32),
                       pltpu.MemorySpace.VMEM_SHARED((AP, 128), jnp.float32),
                       pltpu.SemaphoreType.DMA])


@functools.cache
def _scb():
    return pl.kernel(_scb_body, **_scb_kw())


def _t3_body(s_ref, wm1_ref, bm1_ref, wm2_ref, bm2_ref,
             ws1_ref, bs1_ref, ws2_ref, bs2_ref, wa1_ref, ba1_ref, wa2_ref,
             wh_ref, bh_ref,
             qf_ref, qg_ref, qd_ref, zf_ref, mu_ref, xs_ref, zd_ref, wts_ref):
    dn = (((1,), (1,)), ((), ()))

    def lin(a, w_ref, b_ref=None):
        y = lax.dot_general(a, w_ref[...], dn,
                            preferred_element_type=jnp.float32)
        if b_ref is not None:
            y = y + b_ref[...]
        return y

    t = s_ref[0] + s_ref[1]
    h1 = jax.nn.relu(jnp.concatenate([t[q][:, :64] for q in range(4)], axis=1))
    h2 = jax.nn.relu(jnp.concatenate([t[q][:, 64:] for q in range(4)], axis=1))

    mu = lin(jax.nn.relu(lin(h1, wm1_ref, bm1_ref)), wm2_ref, bm2_ref)
    xstd = jax.nn.softplus(
        lin(jax.nn.relu(lin(h1, ws1_ref, bs1_ref)), ws2_ref, bs2_ref))
    zden = lin(jax.nn.relu(lin(h2, wm1_ref, bm1_ref)), wm2_ref, bm2_ref)

    sg = lin(jnp.tanh(lin(mu, wa1_ref, ba1_ref)), wa2_ref) * (1.0 / TEMP)
    sd = lin(jnp.tanh(lin(zden, wa1_ref, ba1_ref)), wa2_ref) * (1.0 / TEMP)
    mx = jnp.maximum(sg, sd)
    eg = jnp.exp(sg - mx)
    ed = jnp.exp(sd - mx)
    tot = eg + ed
    wg_ = eg / tot
    wd_ = ed / tot
    zf = wg_ * mu + wd_ * zden

    qf_ref[...] = jax.nn.softmax(lin(zf, wh_ref, bh_ref), axis=1)
    qg_ref[...] = jax.nn.softmax(lin(mu, wh_ref, bh_ref), axis=1)
    qd_ref[...] = jax.nn.softmax(lin(zden, wh_ref, bh_ref), axis=1)
    zf_ref[...] = zf
    mu_ref[...] = mu
    xs_ref[...] = xstd
    zd_ref[...] = zden
    wts_ref[...] = jnp.concatenate([wg_, wd_], axis=1)


def _t3(sp, Wm1, bm1, Wm2, bm2, Ws1, bs1, Ws2, bs2,
        Wa1, ba1, Wa2, Wh, bh):
    bn = 2000
    grid = (N // bn,)
    full = lambda a: pl.BlockSpec(a.shape, lambda i: (0,) * a.ndim)
    ws = [Wm1, bm1, Wm2, bm2, Ws1, bs1, Ws2, bs2, Wa1, ba1, Wa2, Wh, bh]
    return pl.pallas_call(
        _t3_body,
        grid=grid,
        in_specs=[pl.BlockSpec((NC, 4, bn, 128), lambda i: (0, 0, i, 0))]
                 + [full(w) for w in ws],
        out_specs=[pl.BlockSpec((bn, C), lambda i: (i, 0))] * 3
                  + [pl.BlockSpec((bn, Z), lambda i: (i, 0))] * 4
                  + [pl.BlockSpec((bn, 2), lambda i: (i, 0))],
        out_shape=[jax.ShapeDtypeStruct((N, C), jnp.float32)] * 3
                  + [jax.ShapeDtypeStruct((N, Z), jnp.float32)] * 4
                  + [jax.ShapeDtypeStruct((N, 2), jnp.float32)],
    )(sp, *ws)


def _t4_body(zi_ref, zj_ref, out_ref):
    out_ref[...] = lax.dot_general(
        zi_ref[...], zj_ref[...], (((1,), (1,)), ((), ())),
        preferred_element_type=jnp.float32)


def _t4(z):
    bi, bj = 2000, 1024
    return pl.pallas_call(
        _t4_body,
        grid=(N // bi, pl.cdiv(N, bj)),
        in_specs=[pl.BlockSpec((bi, Z), lambda i, j: (i, 0)),
                  pl.BlockSpec((bj, Z), lambda i, j: (j, 0))],
        out_specs=pl.BlockSpec((bi, bj), lambda i, j: (i, j)),
        out_shape=jax.ShapeDtypeStruct((N, N), jnp.float32),
    )(z, z)


def kernel(x, edge_index, adj_values, Wg, bg, Wm1, bm1, Wm2, bm2,
           Ws1, bs1, Ws2, bs2, Wnb, bnb, Wsf, bsf, Watt, batt,
           Wa1, ba1, Wa2, Wh, bh):
    rows = edge_index[0]
    cols = edge_index[1]
    pad3 = lambda a: jnp.pad(a.reshape(NW, ET),
                             ((0, 0), (0, ETP - ET))).reshape(NW, EJ, 128)
    r3 = pad3(rows)
    c3 = pad3(cols)
    a3 = pad3(adj_values)

    hm4, uv = _t1(x, Wg, bg, Wnb, bnb, Wsf, bsf, Watt)
    u = uv[:, 0] + 0.0
    v = uv[:, 1] + 0.0

    zn = jnp.zeros((RST,), jnp.float32)
    b16 = jnp.full((16,), batt[0], jnp.float32)
    mv3, rs, l0p = _sca()(r3, c3, a3, u, v, zn, b16)

    dinv2, l0m = _tmid(rs, l0p)
    dinv = dinv2.reshape(RSP)

    nv3 = _scn()(r3, c3, mv3, dinv)

    za = jnp.zeros((AT, 128), jnp.float32)
    hq = tuple(hm4[q] + 0.0 for q in range(4))
    sp = _scb()(r3, c3, a3, nv3, *hq, za)

    q_fused, q_gen, q_den, z_fused, mu, x_std, z_den, wts = _t3(
        sp, Wm1, bm1, Wm2, bm2, Ws1, bs1, Ws2, bs2,
        Wa1, ba1, Wa2, Wh, bh)

    adj_logits = _t4(mu)
    l0 = l0m.reshape(())
    return (q_fused, q_gen, q_den, adj_logits, z_fused, mu, x_std, l0,
            z_den, wts[:, :, None])
